# Initial kernel scaffold; baseline (speedup 1.0000x reference)
import jax
import jax.numpy as jnp
from jax.experimental import pallas as pl


def kernel(relation_indices, relation_embedding_weight):
    raise NotImplementedError("write your pallas kernel here")


# speedup vs baseline: 1.3088x; 1.3088x over previous
import functools

import jax
import jax.numpy as jnp
from jax import lax
from jax.experimental import pallas as pl
from jax.experimental.pallas import tpu as pltpu
from jax.experimental.pallas import tpu_sc as plsc

B = 4096
L = 200
D = 32
N = B * L
NC = 2
NS = 16
NW = NC * NS
PER_W = N // NW
CHUNK = 128
NCH = PER_W // CHUNK


def _body(idx_hbm, table_hbm, out_hbm, idx_v, rows_v, gsem):
    cid = lax.axis_index("c")
    sid = lax.axis_index("s")
    wid = sid * NC + cid
    pltpu.sync_copy(idx_hbm.at[wid], idx_v)
    base = wid * PER_W

    def step(j, _):
        pltpu.async_copy(table_hbm.at[idx_v.at[j]], rows_v, gsem).wait()
        pltpu.sync_copy(rows_v, out_hbm.at[pl.ds(base + j * CHUNK, CHUNK)])
        return 0

    lax.fori_loop(0, NCH, step, 0)


@jax.jit
def _gather(rel_idx, table):
    run = pl.kernel(
        _body,
        mesh=plsc.VectorSubcoreMesh(core_axis_name="c", subcore_axis_name="s"),
        out_type=jax.ShapeDtypeStruct((N, D), jnp.float32),
        scratch_types=[
            pltpu.VMEM((NCH, CHUNK), jnp.int32),
            pltpu.VMEM((CHUNK, D), jnp.float32),
            pltpu.SemaphoreType.DMA,
        ],
        compiler_params=pltpu.CompilerParams(use_tc_tiling_on_sc=False),
    )
    return run(rel_idx, table)


def kernel(relation_indices, relation_embedding_weight):
    rel_idx = relation_indices[..., 0].reshape(NW, NCH, CHUNK)
    out = _gather(rel_idx, relation_embedding_weight)
    return out.reshape(B, L, D)


# speedup vs baseline: 1.4989x; 1.1452x over previous
import jax
import jax.numpy as jnp
from jax import lax
from jax.experimental import pallas as pl
from jax.experimental.pallas import tpu as pltpu
from jax.experimental.pallas import tpu_sc as plsc

B = 4096
L = 200
D = 32
N = B * L
NC = 2
NS = 16
NW = NC * NS
PER_W = N // NW
CHUNK = 128
NCH = PER_W // CHUNK
NBUF = 8
K = 4


def _body(idx_hbm, table_hbm, out_hbm, idx_v, rows_v, gsem, wsem):
    cid = lax.axis_index("c")
    sid = lax.axis_index("s")
    wid = sid * NC + cid
    pltpu.sync_copy(idx_hbm.at[wid], idx_v)
    base = wid * PER_W

    def start_gather(j, b):
        pltpu.async_copy(table_hbm.at[idx_v.at[j]], rows_v.at[b], gsem.at[b])

    def wait_gather(b):
        pltpu.make_async_copy(table_hbm.at[idx_v.at[0]], rows_v.at[b],
                              gsem.at[b]).wait()

    def start_wb(j, b):
        pltpu.async_copy(rows_v.at[b], out_hbm.at[pl.ds(base + j * CHUNK, CHUNK)],
                         wsem.at[b])

    def wait_wb(b):
        pltpu.make_async_copy(rows_v.at[b], out_hbm.at[pl.ds(base, CHUNK)],
                              wsem.at[b]).wait()

    for j in range(K):
        start_gather(j, j)
    for j in range(NBUF - K):
        wait_gather(j % NBUF)
        start_wb(j, j % NBUF)
        start_gather(j + K, (j + K) % NBUF)

    ROUNDS = (NCH - NBUF) // NBUF

    def round_body(g, _):
        j0 = (NBUF - K) + g * NBUF
        for b in range(NBUF):
            j = j0 + b
            bi = (NBUF - K + b) % NBUF
            bb = b
            wait_gather(bi)
            start_wb(j, bi)
            wait_wb(bb)
            start_gather(j + K, bb)
        return 0

    lax.fori_loop(0, ROUNDS, round_body, 0)

    for j in range(NCH - K, NCH):
        wait_gather(j % NBUF)
        start_wb(j, j % NBUF)
    for b in range(NBUF):
        wait_wb(b)


@jax.jit
def _gather(rel_idx, table):
    run = pl.kernel(
        _body,
        mesh=plsc.VectorSubcoreMesh(core_axis_name="c", subcore_axis_name="s"),
        out_type=jax.ShapeDtypeStruct((N, D), jnp.float32),
        scratch_types=[
            pltpu.VMEM((NCH, CHUNK), jnp.int32),
            pltpu.VMEM((NBUF, CHUNK, D), jnp.float32),
            pltpu.SemaphoreType.DMA((NBUF,)),
            pltpu.SemaphoreType.DMA((NBUF,)),
        ],
        compiler_params=pltpu.CompilerParams(use_tc_tiling_on_sc=False),
    )
    return run(rel_idx, table)


def kernel(relation_indices, relation_embedding_weight):
    rel_idx = relation_indices[..., 0].reshape(NW, NCH, CHUNK)
    out = _gather(rel_idx, relation_embedding_weight)
    return out.reshape(B, L, D)
